# baseline (speedup 1.0000x reference)
import jax
import jax.numpy as jnp
from jax import lax
from jax.experimental import pallas as pl
from jax.experimental.pallas import tpu as pltpu
from jax.experimental.pallas import tpu_sc as plsc

VOCAB = 1000000
EMBED = 32
N = 16384
T = 50

_info = plsc.get_sparse_core_info()
NC, NS = _info.num_cores, _info.num_subcores
NW = NC * NS
N_PER_W = N // NW
UB = N_PER_W // 128
NBLK = UB * T


def _embed_body(xt_hbm, table_hbm, out_hbm, idx_v, grows, trows, gs0, gs1, os0, os1):
    gsems = (gs0, gs1)
    osems = (os0, os1)
    wid = lax.axis_index("s") * NC + lax.axis_index("c")
    nbase = wid * N_PER_W
    ubase = wid * UB

    pltpu.sync_copy(xt_hbm.at[pl.ds(0, T), pl.ds(nbase, N_PER_W)], idx_v)

    iota16 = lax.iota(jnp.int32, 16)
    rows16 = [iota16 + 16 * j for j in range(8)]

    def idx_list(b):
        u_loc = b // T
        t = b - u_loc * T
        return idx_v.at[t, pl.ds(u_loc * 128, 128)], t, u_loc

    def fire_gather(b, h):
        il, _, _ = idx_list(b)
        pltpu.async_copy(table_hbm.at[il], grows.at[h], gsems[h])

    def drain_gather(b, h):
        il, _, _ = idx_list(b)
        pltpu.make_async_copy(table_hbm.at[il], grows.at[h], gsems[h]).wait()

    def transpose(h):
        @pl.loop(0, EMBED, unroll=8)
        def _cols(e):
            cols = jnp.full((16,), e, jnp.int32)
            for j in range(8):
                vals = plsc.load_gather(grows.at[h], [rows16[j], cols])
                trows[h, e, pl.ds(16 * j, 16)] = vals

    def fire_out(b, h):
        _, t, u_loc = idx_list(b)
        u = ubase + u_loc
        for g in range(4):
            pltpu.async_copy(
                trows.at[h, pl.ds(8 * g, 8)], out_hbm.at[t, g, u], osems[h]
            )

    def drain_out(b, h):
        _, t, u_loc = idx_list(b)
        u = ubase + u_loc
        for g in range(4):
            pltpu.make_async_copy(
                trows.at[h, pl.ds(8 * g, 8)], out_hbm.at[t, g, u], osems[h]
            ).wait()

    fire_gather(0, 0)
    fire_gather(1, 1)
    for b in range(2):
        drain_gather(b, b)
        transpose(b)
        fire_out(b, b)
        fire_gather(b + 2, b)

    @pl.loop(0, (NBLK - 4) // 2)
    def _steady(gg):
        for h in range(2):
            b = 2 + 2 * gg + h
            drain_out(b - 2, h)
            drain_gather(b, h)
            transpose(h)
            fire_out(b, h)
            fire_gather(b + 2, h)

    for h in range(2):
        b = NBLK - 2 + h
        drain_out(b - 2, h)
        drain_gather(b, h)
        transpose(h)
        fire_out(b, h)
    for h in range(2):
        drain_out(NBLK - 2 + h, h)


def kernel(x, W_embed):
    mesh = plsc.VectorSubcoreMesh(core_axis_name="c", subcore_axis_name="s")
    fn = pl.kernel(
        _embed_body,
        out_type=jax.ShapeDtypeStruct((T, EMBED // 8, N // 128, 8, 128), jnp.float32),
        mesh=mesh,
        scratch_types=[
            pltpu.VMEM((T, N_PER_W), jnp.int32),
            pltpu.VMEM((2, 128, EMBED), jnp.float32),
            pltpu.VMEM((2, EMBED, 128), jnp.float32),
            pltpu.SemaphoreType.DMA,
            pltpu.SemaphoreType.DMA,
            pltpu.SemaphoreType.DMA,
            pltpu.SemaphoreType.DMA,
        ],
        compiler_params=pltpu.CompilerParams(
            use_tc_tiling_on_sc=False, needs_layout_passes=False
        ),
    )
    out5d = fn(x.T.astype(jnp.int32), W_embed)
    return out5d.transpose(2, 4, 0, 1, 3).reshape(N, T, EMBED)


# speedup vs baseline: 1.6549x; 1.6549x over previous
import jax
import jax.numpy as jnp
from jax import lax
from jax.experimental import pallas as pl
from jax.experimental.pallas import tpu as pltpu
from jax.experimental.pallas import tpu_sc as plsc

VOCAB = 1000000
EMBED = 32
N = 16384
T = 50

_info = plsc.get_sparse_core_info()
NC, NS = _info.num_cores, _info.num_subcores
NW = NC * NS
N_PER_W = N // NW
UB = N_PER_W // 128
NBLK = UB * T


def _embed_body(xt_hbm, table_hbm, out_hbm, idx_v, grows, trows, gs0, gs1, os0, os1):
    gsems = (gs0, gs1)
    osems = (os0, os1)
    wid = lax.axis_index("s") * NC + lax.axis_index("c")
    nbase = wid * N_PER_W
    ubase = wid * UB

    pltpu.sync_copy(xt_hbm.at[pl.ds(0, T), pl.ds(nbase, N_PER_W)], idx_v)

    iota16 = lax.iota(jnp.int32, 16)
    col16 = [(iota16 + k) & 15 for k in range(16)]

    def idx_list(b):
        u_loc = b // T
        t = b - u_loc * T
        return idx_v.at[t, pl.ds(u_loc * 128, 128)], t, u_loc

    def fire_gather(b, h):
        il, _, _ = idx_list(b)
        pltpu.async_copy(table_hbm.at[il], grows.at[h], gsems[h])

    def drain_gather(b, h):
        il, _, _ = idx_list(b)
        pltpu.make_async_copy(table_hbm.at[il], grows.at[h], gsems[h]).wait()

    def transpose(h):
        @pl.loop(0, 8)
        def _j(j):
            l_ids = iota16 + 16 * j
            for m in range(2):
                for k in range(16):
                    e_ids = col16[k] + 16 * m
                    vals = plsc.load_gather(grows.at[h], [l_ids, e_ids])
                    plsc.store_scatter(trows.at[h], [e_ids, l_ids], vals)

    def fire_out(b, h):
        _, t, u_loc = idx_list(b)
        u = ubase + u_loc
        for g in range(4):
            pltpu.async_copy(
                trows.at[h, pl.ds(8 * g, 8)], out_hbm.at[t, g, u], osems[h]
            )

    def drain_out(b, h):
        _, t, u_loc = idx_list(b)
        u = ubase + u_loc
        for g in range(4):
            pltpu.make_async_copy(
                trows.at[h, pl.ds(8 * g, 8)], out_hbm.at[t, g, u], osems[h]
            ).wait()

    fire_gather(0, 0)
    fire_gather(1, 1)
    for b in range(2):
        drain_gather(b, b)
        transpose(b)
        fire_out(b, b)
        fire_gather(b + 2, b)

    @pl.loop(0, (NBLK - 4) // 2)
    def _steady(gg):
        for h in range(2):
            b = 2 + 2 * gg + h
            drain_out(b - 2, h)
            drain_gather(b, h)
            transpose(h)
            fire_out(b, h)
            fire_gather(b + 2, h)

    for h in range(2):
        b = NBLK - 2 + h
        drain_out(b - 2, h)
        drain_gather(b, h)
        transpose(h)
        fire_out(b, h)
    for h in range(2):
        drain_out(NBLK - 2 + h, h)


def kernel(x, W_embed):
    mesh = plsc.VectorSubcoreMesh(core_axis_name="c", subcore_axis_name="s")
    fn = pl.kernel(
        _embed_body,
        out_type=jax.ShapeDtypeStruct((T, EMBED // 8, N // 128, 8, 128), jnp.float32),
        mesh=mesh,
        scratch_types=[
            pltpu.VMEM((T, N_PER_W), jnp.int32),
            pltpu.VMEM((2, 128, EMBED), jnp.float32),
            pltpu.VMEM((2, EMBED, 128), jnp.float32),
            pltpu.SemaphoreType.DMA,
            pltpu.SemaphoreType.DMA,
            pltpu.SemaphoreType.DMA,
            pltpu.SemaphoreType.DMA,
        ],
        compiler_params=pltpu.CompilerParams(
            use_tc_tiling_on_sc=False, needs_layout_passes=False
        ),
    )
    out5d = fn(x.T.astype(jnp.int32), W_embed)
    return out5d.transpose(2, 4, 0, 1, 3).reshape(N, T, EMBED)
